# baseline (speedup 1.0000x reference)
import functools

import jax
import jax.numpy as jnp
from jax import lax
from jax.experimental import pallas as pl
from jax.experimental.pallas import tpu as pltpu
from jax.experimental.pallas import tpu_sc as plsc

N = 10000
E = 320000

_NC = 2
_NS = 16
_K = 40
_CPW = E // (_NC * _NS * _K)
_RPT = 640
_NB = 5
_KD = 80
_CPWD = E // (_NC * _NS * _KD)


def _sc_edge_agg(table, src2d, dst2d, D, K, NB=_NB, tc_tiling=False):
    cpw = E // (_NC * _NS * K)
    nb = min(NB, cpw)
    mesh = plsc.VectorSubcoreMesh(core_axis_name="c", subcore_axis_name="s")

    @functools.partial(
        pl.kernel,
        out_type=jax.ShapeDtypeStruct((_NC, N, D), jnp.float32),
        mesh=mesh,
        compiler_params=pltpu.CompilerParams(use_tc_tiling_on_sc=tc_tiling),
        scratch_types=[
            pltpu.VMEM_SHARED((N, D), jnp.float32),
            pltpu.VMEM((cpw, K), jnp.int32),
            pltpu.VMEM((cpw, K), jnp.int32),
            pltpu.VMEM((nb, K, D), jnp.float32),
            pltpu.SemaphoreType.DMA((nb,)),
            pltpu.SemaphoreType.DMA((nb,)),
        ],
        name=f"sc_edge_agg_d{D}_k{K}_nb{NB}",
    )
    def k(table_h, src_h, dst_h, out_h, accum, src_idx, dst_idx, rows, gsem, ssem):
        c = lax.axis_index("c")
        s = lax.axis_index("s")
        w = c * _NS + s

        def gather_start(i, b):
            pltpu.async_copy(table_h.at[src_idx.at[i]], rows.at[b], gsem.at[b])

        def gather_wait(i, b):
            pltpu.make_async_copy(table_h.at[src_idx.at[i]], rows.at[b],
                                  gsem.at[b]).wait()

        def scatter_start(i, b):
            pltpu.async_copy(rows.at[b], accum.at[dst_idx.at[i]], ssem.at[b],
                             add=True)

        def scatter_wait(i, b):
            pltpu.make_async_copy(rows.at[b], accum.at[dst_idx.at[i]],
                                  ssem.at[b]).wait()

        @pl.when(s < _NS - 1)
        def _():
            pltpu.async_copy(table_h.at[pl.ds(s * _RPT, _RPT)],
                             accum.at[pl.ds(s * _RPT, _RPT)], gsem.at[0])

        @pl.when(s == _NS - 1)
        def _():
            pltpu.async_copy(table_h.at[pl.ds((_NS - 1) * _RPT, N - (_NS - 1) * _RPT)],
                             accum.at[pl.ds((_NS - 1) * _RPT, N - (_NS - 1) * _RPT)],
                             gsem.at[0])

        pltpu.async_copy(src_h.at[w], src_idx, ssem.at[0])
        pltpu.async_copy(dst_h.at[w], dst_idx, ssem.at[1])

        @pl.when(s < _NS - 1)
        def _():
            pltpu.make_async_copy(table_h.at[pl.ds(s * _RPT, _RPT)],
                                  accum.at[pl.ds(s * _RPT, _RPT)],
                                  gsem.at[0]).wait()

        @pl.when(s == _NS - 1)
        def _():
            pltpu.make_async_copy(
                table_h.at[pl.ds((_NS - 1) * _RPT, N - (_NS - 1) * _RPT)],
                accum.at[pl.ds((_NS - 1) * _RPT, N - (_NS - 1) * _RPT)],
                gsem.at[0]).wait()

        pltpu.make_async_copy(src_h.at[w], src_idx, ssem.at[0]).wait()
        pltpu.make_async_copy(dst_h.at[w], dst_idx, ssem.at[1]).wait()

        plsc.subcore_barrier()

        for b in range(nb):
            gather_start(b, b)

        def group(o, carry):
            for b in range(nb):
                i = o * nb + b
                gather_wait(i, b)
                scatter_start(i, b)
            for b in range(nb):
                i = o * nb + b
                scatter_wait(i, b)
                gather_start(i + nb, b)
            return carry

        n_groups = cpw // nb - 1
        lax.fori_loop(0, n_groups, group, 0, unroll=False)

        tail_start = n_groups * nb
        for i in range(tail_start, cpw):
            b = i % nb
            gather_wait(i, b)
            scatter_start(i, b)
            nxt = i + nb
            if nxt < cpw:
                scatter_wait(i, b)
                gather_start(nxt, b)
        for i in range(max(tail_start, cpw - nb), cpw):
            scatter_wait(i, i % nb)

        plsc.subcore_barrier()

        @pl.when(s < _NS - 1)
        def _():
            pltpu.sync_copy(accum.at[pl.ds(s * _RPT, _RPT)],
                            out_h.at[c].at[pl.ds(s * _RPT, _RPT)])

        @pl.when(s == _NS - 1)
        def _():
            pltpu.sync_copy(accum.at[pl.ds((_NS - 1) * _RPT, N - (_NS - 1) * _RPT)],
                            out_h.at[c].at[pl.ds((_NS - 1) * _RPT, N - (_NS - 1) * _RPT)])

    return k(table, src2d, dst2d)


def _sc_edge_agg_ring(table, idx4, D, K, NB):
    cpw = E // (_NC * _NS * K)
    nb = NB
    mesh = plsc.VectorSubcoreMesh(core_axis_name="c", subcore_axis_name="s")

    @functools.partial(
        pl.kernel,
        out_type=jax.ShapeDtypeStruct((_NC, N, D), jnp.float32),
        mesh=mesh,
        compiler_params=pltpu.CompilerParams(use_tc_tiling_on_sc=False),
        scratch_types=[
            pltpu.VMEM_SHARED((N, D), jnp.float32),
            pltpu.VMEM((nb, 2, K), jnp.int32),
            pltpu.VMEM((nb, K, D), jnp.float32),
            pltpu.SemaphoreType.DMA((nb,)),
            pltpu.SemaphoreType.DMA((nb,)),
            pltpu.SemaphoreType.DMA((nb,)),
        ],
        name=f"sc_edge_agg_ring_d{D}_k{K}_nb{NB}",
    )
    def k(table_h, idx_h, out_h, accum, iring, rows, isem, gsem, ssem):
        c = lax.axis_index("c")
        s = lax.axis_index("s")
        w = c * _NS + s

        def iload(i, b):
            pltpu.async_copy(idx_h.at[w, i], iring.at[b], isem.at[b])

        def iwait(i, b):
            pltpu.make_async_copy(idx_h.at[w, i], iring.at[b],
                                  isem.at[b]).wait()

        def gather_start(i, b):
            pltpu.async_copy(table_h.at[iring.at[b, 0]], rows.at[b],
                             gsem.at[b])

        def gather_wait(i, b):
            pltpu.make_async_copy(table_h.at[iring.at[b, 0]], rows.at[b],
                                  gsem.at[b]).wait()

        def scatter_start(i, b):
            pltpu.async_copy(rows.at[b], accum.at[iring.at[b, 1]], ssem.at[b],
                             add=True)

        def scatter_wait(i, b):
            pltpu.make_async_copy(rows.at[b], accum.at[iring.at[b, 1]],
                                  ssem.at[b]).wait()

        @pl.when(s < _NS - 1)
        def _():
            pltpu.async_copy(table_h.at[pl.ds(s * _RPT, _RPT)],
                             accum.at[pl.ds(s * _RPT, _RPT)], gsem.at[0])

        @pl.when(s == _NS - 1)
        def _():
            pltpu.async_copy(table_h.at[pl.ds((_NS - 1) * _RPT, N - (_NS - 1) * _RPT)],
                             accum.at[pl.ds((_NS - 1) * _RPT, N - (_NS - 1) * _RPT)],
                             gsem.at[0])

        for b in range(nb):
            iload(b, b)

        @pl.when(s < _NS - 1)
        def _():
            pltpu.make_async_copy(table_h.at[pl.ds(s * _RPT, _RPT)],
                                  accum.at[pl.ds(s * _RPT, _RPT)],
                                  gsem.at[0]).wait()

        @pl.when(s == _NS - 1)
        def _():
            pltpu.make_async_copy(
                table_h.at[pl.ds((_NS - 1) * _RPT, N - (_NS - 1) * _RPT)],
                accum.at[pl.ds((_NS - 1) * _RPT, N - (_NS - 1) * _RPT)],
                gsem.at[0]).wait()

        plsc.subcore_barrier()

        def group(o, carry):
            for b in range(nb):
                i = o * nb + b
                iwait(i, b)
                gather_start(i, b)
            for b in range(nb):
                i = o * nb + b
                gather_wait(i, b)
                scatter_start(i, b)
            for b in range(nb):
                i = o * nb + b
                scatter_wait(i, b)
                iload(i + nb, b)
            return carry

        n_groups = cpw // nb - 1
        lax.fori_loop(0, n_groups, group, 0, unroll=False)

        tail_start = n_groups * nb
        waited = set()
        for i in range(tail_start, cpw):
            b = i % nb
            if i >= tail_start + nb:
                scatter_wait(i - nb, b)
                waited.add(i - nb)
                iload(i, b)
            iwait(i, b)
            gather_start(i, b)
            gather_wait(i, b)
            scatter_start(i, b)
        for i in range(tail_start, cpw):
            if i not in waited:
                scatter_wait(i, i % nb)

        plsc.subcore_barrier()

        @pl.when(s < _NS - 1)
        def _():
            pltpu.sync_copy(accum.at[pl.ds(s * _RPT, _RPT)],
                            out_h.at[c].at[pl.ds(s * _RPT, _RPT)])

        @pl.when(s == _NS - 1)
        def _():
            pltpu.sync_copy(accum.at[pl.ds((_NS - 1) * _RPT, N - (_NS - 1) * _RPT)],
                            out_h.at[c].at[pl.ds((_NS - 1) * _RPT, N - (_NS - 1) * _RPT)])

    return k(table, idx4)


def _sc_degree(dst3d):
    mesh = plsc.VectorSubcoreMesh(core_axis_name="c", subcore_axis_name="s")

    @functools.partial(
        pl.kernel,
        out_type=jax.ShapeDtypeStruct((_NC, N), jnp.float32),
        mesh=mesh,
        scratch_types=[
            pltpu.VMEM_SHARED((N,), jnp.float32),
            pltpu.VMEM((_CPWD, _KD), jnp.int32),
            pltpu.VMEM((_KD,), jnp.float32),
            pltpu.VMEM((_RPT,), jnp.float32),
            pltpu.SemaphoreType.DMA((8,)),
        ],
        compiler_params=pltpu.CompilerParams(use_tc_tiling_on_sc=False),
        name="sc_degree",
    )
    def k(dst_h, out_h, accum, dst_idx, ones_v, zbuf, dsem):
        c = lax.axis_index("c")
        s = lax.axis_index("s")
        w = c * _NS + s

        def fill(i, carry):
            zbuf[pl.ds(i * 16, 16)] = jnp.zeros((16,), jnp.float32)
            return carry
        lax.fori_loop(0, _RPT // 16, fill, 0)

        def fill1(i, carry):
            ones_v[pl.ds(i * 16, 16)] = jnp.ones((16,), jnp.float32)
            return carry
        lax.fori_loop(0, _KD // 16, fill1, 0)

        @pl.when(s < _NS - 1)
        def _():
            pltpu.sync_copy(zbuf, accum.at[pl.ds(s * _RPT, _RPT)])

        @pl.when(s == _NS - 1)
        def _():
            pltpu.sync_copy(zbuf.at[pl.ds(0, N - (_NS - 1) * _RPT)],
                            accum.at[pl.ds((_NS - 1) * _RPT, N - (_NS - 1) * _RPT)])

        pltpu.sync_copy(dst_h.at[w], dst_idx)
        plsc.subcore_barrier()

        def dstart(i, b):
            pltpu.async_copy(ones_v, accum.at[dst_idx.at[i]], dsem.at[b],
                             add=True)

        def dwait(i, b):
            pltpu.make_async_copy(ones_v, accum.at[dst_idx.at[i]],
                                  dsem.at[b]).wait()

        for b in range(8):
            dstart(b, b)

        def grp(o, carry):
            for b in range(8):
                i = o * 8 + b
                dwait(i, b)
                dstart(i + 8, b)
            return carry
        ng = _CPWD // 8 - 1
        lax.fori_loop(0, ng, grp, 0, unroll=False)
        for i in range(ng * 8, _CPWD):
            b = i % 8
            dwait(i, b)
            nxt = i + 8
            if nxt < _CPWD:
                dstart(nxt, b)

        plsc.subcore_barrier()

        @pl.when(s < _NS - 1)
        def _():
            pltpu.sync_copy(accum.at[pl.ds(s * _RPT, _RPT)],
                            out_h.at[c].at[pl.ds(s * _RPT, _RPT)])

        @pl.when(s == _NS - 1)
        def _():
            pltpu.sync_copy(accum.at[pl.ds((_NS - 1) * _RPT, N - (_NS - 1) * _RPT)],
                            out_h.at[c].at[pl.ds((_NS - 1) * _RPT, N - (_NS - 1) * _RPT)])

    return k(dst3d)


_BLK = 2000
_G = N // _BLK

def _rows(d):
    return pl.BlockSpec((_BLK, d), lambda i: (i, 0))

def _part(d):
    return pl.BlockSpec((_NC, _BLK, d), lambda i: (0, i, 0))

def _full(a, b):
    return pl.BlockSpec((a, b), lambda i: (0, 0))

_DOT = dict(preferred_element_type=jnp.float32, precision=lax.Precision.HIGHEST)


def _tc_h0(x, W_in, b_in):
    def body(x_r, w_r, b_r, o_r):
        o_r[...] = jnp.dot(x_r[...], w_r[...], **_DOT) + b_r[...]
    return pl.pallas_call(
        body, grid=(_G,),
        in_specs=[_rows(128), _full(128, 128), _full(1, 128)],
        out_specs=_rows(128),
        out_shape=jax.ShapeDtypeStruct((N, 128), jnp.float32),
    )(x, W_in, b_in)


def _tc_g0(degP, h0, Wg0):
    def body(dp_r, h_r, w_r, dinv_r, g_r):
        deg = dp_r[0] + dp_r[1] + 1.0
        dinv = lax.rsqrt(deg)
        dinv_r[...] = dinv
        g_r[...] = dinv * jnp.dot(h_r[...], w_r[...], **_DOT)
    return pl.pallas_call(
        body, grid=(_G,),
        in_specs=[_part(1), _rows(128), _full(128, 128)],
        out_specs=[_rows(1), _rows(128)],
        out_shape=[jax.ShapeDtypeStruct((N, 1), jnp.float32),
                   jax.ShapeDtypeStruct((N, 128), jnp.float32)],
    )(degP, h0, Wg0)


def _tc_layer(P, g, dinv, bg, Wn, dout, want_h=True):
    def body(p_r, g_r, di_r, b_r, w_r, *outs):
        s = p_r[0] + p_r[1] - g_r[...]
        pre = di_r[...] * s + b_r[...]
        h = jnp.where(pre > 0, pre, 0.2 * pre)
        if want_h:
            outs[0][...] = h
        outs[-1][...] = di_r[...] * jnp.dot(h, w_r[...], **_DOT)
    out_specs = [_rows(128), _rows(dout)] if want_h else [_rows(dout)]
    out_shape = ([jax.ShapeDtypeStruct((N, 128), jnp.float32)] if want_h else []) + [
        jax.ShapeDtypeStruct((N, dout), jnp.float32)]
    res = pl.pallas_call(
        body, grid=(_G,),
        in_specs=[_part(128), _rows(128), _rows(1), _full(1, 128),
                  _full(128, dout)],
        out_specs=out_specs,
        out_shape=out_shape,
    )(P, g, dinv, bg, Wn)
    return res if want_h else (None, res[0])


def _tc_skips(h0, h1, h2, Ws0, bs0, Ws1, bs1, Ws2, bs2):
    def body(a_r, b_r, c_r, w0, v0, w1, v1, w2, v2, o_r):
        o_r[...] = (jnp.dot(a_r[...], w0[...], **_DOT) + v0[...]
                    + jnp.dot(b_r[...], w1[...], **_DOT) + v1[...]
                    + jnp.dot(c_r[...], w2[...], **_DOT) + v2[...])
    return pl.pallas_call(
        body, grid=(_G,),
        in_specs=[_rows(128), _rows(128), _rows(128),
                  _full(128, 64), _full(1, 64),
                  _full(128, 64), _full(1, 64),
                  _full(128, 64), _full(1, 64)],
        out_specs=_rows(64),
        out_shape=jax.ShapeDtypeStruct((N, 64), jnp.float32),
    )(h0, h1, h2, Ws0, bs0, Ws1, bs1, Ws2, bs2)


def _tc_final(PL, gl, dinv, b_lat, skips):
    def body(p_r, g_r, di_r, b_r, sk_r, o_r):
        s = p_r[0] + p_r[1] - g_r[...]
        o_r[...] = di_r[...] * s + b_r[...] + sk_r[...]
    return pl.pallas_call(
        body, grid=(_G,),
        in_specs=[_part(64), _rows(64), _rows(1), _full(1, 64), _rows(64)],
        out_specs=_rows(64),
        out_shape=jax.ShapeDtypeStruct((N, 64), jnp.float32),
    )(PL, gl, dinv, b_lat, skips)


def kernel(x, edge_index, W_in, b_in, Wg0, bg0, Wg1, bg1, Wg2, bg2,
           W_lat, b_lat, Ws0, bs0, Ws1, bs1, Ws2, bs2):
    src40 = edge_index[0].reshape(_NC * _NS, E // (_NC * _NS * 40), 40)
    dst40 = edge_index[1].reshape(_NC * _NS, E // (_NC * _NS * 40), 40)
    src80 = edge_index[0].reshape(_NC * _NS, E // (_NC * _NS * 80), 80)
    dst80 = edge_index[1].reshape(_NC * _NS, E // (_NC * _NS * 80), 80)

    idx4 = edge_index.reshape(2, _NC * _NS, E // (_NC * _NS * 40), 40).transpose(1, 2, 0, 3)

    degP = _sc_degree(
        edge_index[1].reshape(_NC * _NS, _CPWD, _KD)).reshape(_NC, N, 1)

    h0 = _tc_h0(x, W_in, b_in.reshape(1, 128))
    dinv, g0 = _tc_g0(degP, h0, Wg0)

    P0 = _sc_edge_agg_ring(g0, idx4, 128, 40, NB=9)
    h1, g1 = _tc_layer(P0, g0, dinv, bg0.reshape(1, 128), Wg1, 128)

    P1 = _sc_edge_agg_ring(g1, idx4, 128, 40, NB=9)
    h2, g2 = _tc_layer(P1, g1, dinv, bg1.reshape(1, 128), Wg2, 128)

    P2 = _sc_edge_agg_ring(g2, idx4, 128, 40, NB=9)
    _h3, gl = _tc_layer(P2, g2, dinv, bg2.reshape(1, 128), W_lat, 64, want_h=False)

    PL = _sc_edge_agg(gl, src80, dst80, 64, 80, NB=12, tc_tiling=False)
    skips = _tc_skips(h0, h1, h2, Ws0, bs0.reshape(1, 64),
                      Ws1, bs1.reshape(1, 64), Ws2, bs2.reshape(1, 64))
    return _tc_final(PL, gl, dinv, b_lat.reshape(1, 64), skips)


# speedup vs baseline: 1.2464x; 1.2464x over previous
import functools

import jax
import jax.numpy as jnp
from jax import lax
from jax.experimental import pallas as pl
from jax.experimental.pallas import tpu as pltpu
from jax.experimental.pallas import tpu_sc as plsc

N = 10000
E = 320000

_NC = 2
_NS = 16
_K = 40
_CPW = E // (_NC * _NS * _K)
_RPT = 640
_NB = 5
_KD = 80
_CPWD = E // (_NC * _NS * _KD)


def _sc_edge_agg(table, src2d, dst2d, D, K, NB=_NB, tc_tiling=False):
    cpw = E // (_NC * _NS * K)
    nb = min(NB, cpw)
    mesh = plsc.VectorSubcoreMesh(core_axis_name="c", subcore_axis_name="s")

    @functools.partial(
        pl.kernel,
        out_type=jax.ShapeDtypeStruct((_NC, N, D), jnp.float32),
        mesh=mesh,
        compiler_params=pltpu.CompilerParams(use_tc_tiling_on_sc=tc_tiling),
        scratch_types=[
            pltpu.VMEM_SHARED((N, D), jnp.float32),
            pltpu.VMEM((cpw, K), jnp.int32),
            pltpu.VMEM((cpw, K), jnp.int32),
            pltpu.VMEM((nb, K, D), jnp.float32),
            pltpu.SemaphoreType.DMA((nb,)),
            pltpu.SemaphoreType.DMA((nb,)),
        ],
        name=f"sc_edge_agg_d{D}_k{K}_nb{NB}",
    )
    def k(table_h, src_h, dst_h, out_h, accum, src_idx, dst_idx, rows, gsem, ssem):
        c = lax.axis_index("c")
        s = lax.axis_index("s")
        w = c * _NS + s

        def gather_start(i, b):
            pltpu.async_copy(table_h.at[src_idx.at[i]], rows.at[b], gsem.at[b])

        def gather_wait(i, b):
            pltpu.make_async_copy(table_h.at[src_idx.at[i]], rows.at[b],
                                  gsem.at[b]).wait()

        def scatter_start(i, b):
            pltpu.async_copy(rows.at[b], accum.at[dst_idx.at[i]], ssem.at[b],
                             add=True)

        def scatter_wait(i, b):
            pltpu.make_async_copy(rows.at[b], accum.at[dst_idx.at[i]],
                                  ssem.at[b]).wait()

        @pl.when(s < _NS - 1)
        def _():
            pltpu.async_copy(table_h.at[pl.ds(s * _RPT, _RPT)],
                             accum.at[pl.ds(s * _RPT, _RPT)], gsem.at[0])

        @pl.when(s == _NS - 1)
        def _():
            pltpu.async_copy(table_h.at[pl.ds((_NS - 1) * _RPT, N - (_NS - 1) * _RPT)],
                             accum.at[pl.ds((_NS - 1) * _RPT, N - (_NS - 1) * _RPT)],
                             gsem.at[0])

        pltpu.async_copy(src_h.at[w], src_idx, ssem.at[0])
        pltpu.async_copy(dst_h.at[w], dst_idx, ssem.at[1])

        @pl.when(s < _NS - 1)
        def _():
            pltpu.make_async_copy(table_h.at[pl.ds(s * _RPT, _RPT)],
                                  accum.at[pl.ds(s * _RPT, _RPT)],
                                  gsem.at[0]).wait()

        @pl.when(s == _NS - 1)
        def _():
            pltpu.make_async_copy(
                table_h.at[pl.ds((_NS - 1) * _RPT, N - (_NS - 1) * _RPT)],
                accum.at[pl.ds((_NS - 1) * _RPT, N - (_NS - 1) * _RPT)],
                gsem.at[0]).wait()

        pltpu.make_async_copy(src_h.at[w], src_idx, ssem.at[0]).wait()
        pltpu.make_async_copy(dst_h.at[w], dst_idx, ssem.at[1]).wait()

        plsc.subcore_barrier()

        for b in range(nb):
            gather_start(b, b)

        def group(o, carry):
            for b in range(nb):
                i = o * nb + b
                gather_wait(i, b)
                scatter_start(i, b)
            for b in range(nb):
                i = o * nb + b
                scatter_wait(i, b)
                gather_start(i + nb, b)
            return carry

        n_groups = cpw // nb - 1
        lax.fori_loop(0, n_groups, group, 0, unroll=False)

        tail_start = n_groups * nb
        for i in range(tail_start, cpw):
            b = i % nb
            gather_wait(i, b)
            scatter_start(i, b)
            nxt = i + nb
            if nxt < cpw:
                scatter_wait(i, b)
                gather_start(nxt, b)
        for i in range(max(tail_start, cpw - nb), cpw):
            scatter_wait(i, i % nb)

        plsc.subcore_barrier()

        @pl.when(s < _NS - 1)
        def _():
            pltpu.sync_copy(accum.at[pl.ds(s * _RPT, _RPT)],
                            out_h.at[c].at[pl.ds(s * _RPT, _RPT)])

        @pl.when(s == _NS - 1)
        def _():
            pltpu.sync_copy(accum.at[pl.ds((_NS - 1) * _RPT, N - (_NS - 1) * _RPT)],
                            out_h.at[c].at[pl.ds((_NS - 1) * _RPT, N - (_NS - 1) * _RPT)])

    return k(table, src2d, dst2d)


def _sc_degree(dst3d):
    mesh = plsc.VectorSubcoreMesh(core_axis_name="c", subcore_axis_name="s")

    @functools.partial(
        pl.kernel,
        out_type=jax.ShapeDtypeStruct((_NC, N), jnp.float32),
        mesh=mesh,
        scratch_types=[
            pltpu.VMEM_SHARED((N,), jnp.float32),
            pltpu.VMEM((_CPWD, _KD), jnp.int32),
            pltpu.VMEM((_KD,), jnp.float32),
            pltpu.VMEM((_RPT,), jnp.float32),
            pltpu.SemaphoreType.DMA((8,)),
        ],
        compiler_params=pltpu.CompilerParams(use_tc_tiling_on_sc=False),
        name="sc_degree",
    )
    def k(dst_h, out_h, accum, dst_idx, ones_v, zbuf, dsem):
        c = lax.axis_index("c")
        s = lax.axis_index("s")
        w = c * _NS + s

        def fill(i, carry):
            zbuf[pl.ds(i * 16, 16)] = jnp.zeros((16,), jnp.float32)
            return carry
        lax.fori_loop(0, _RPT // 16, fill, 0)

        def fill1(i, carry):
            ones_v[pl.ds(i * 16, 16)] = jnp.ones((16,), jnp.float32)
            return carry
        lax.fori_loop(0, _KD // 16, fill1, 0)

        @pl.when(s < _NS - 1)
        def _():
            pltpu.sync_copy(zbuf, accum.at[pl.ds(s * _RPT, _RPT)])

        @pl.when(s == _NS - 1)
        def _():
            pltpu.sync_copy(zbuf.at[pl.ds(0, N - (_NS - 1) * _RPT)],
                            accum.at[pl.ds((_NS - 1) * _RPT, N - (_NS - 1) * _RPT)])

        pltpu.sync_copy(dst_h.at[w], dst_idx)
        plsc.subcore_barrier()

        def dstart(i, b):
            pltpu.async_copy(ones_v, accum.at[dst_idx.at[i]], dsem.at[b],
                             add=True)

        def dwait(i, b):
            pltpu.make_async_copy(ones_v, accum.at[dst_idx.at[i]],
                                  dsem.at[b]).wait()

        for b in range(8):
            dstart(b, b)

        def grp(o, carry):
            for b in range(8):
                i = o * 8 + b
                dwait(i, b)
                dstart(i + 8, b)
            return carry
        ng = _CPWD // 8 - 1
        lax.fori_loop(0, ng, grp, 0, unroll=False)
        for i in range(ng * 8, _CPWD):
            b = i % 8
            dwait(i, b)
            nxt = i + 8
            if nxt < _CPWD:
                dstart(nxt, b)

        plsc.subcore_barrier()

        @pl.when(s < _NS - 1)
        def _():
            pltpu.sync_copy(accum.at[pl.ds(s * _RPT, _RPT)],
                            out_h.at[c].at[pl.ds(s * _RPT, _RPT)])

        @pl.when(s == _NS - 1)
        def _():
            pltpu.sync_copy(accum.at[pl.ds((_NS - 1) * _RPT, N - (_NS - 1) * _RPT)],
                            out_h.at[c].at[pl.ds((_NS - 1) * _RPT, N - (_NS - 1) * _RPT)])

    return k(dst3d)


_BLK = 2000
_G = N // _BLK

def _rows(d):
    return pl.BlockSpec((_BLK, d), lambda i: (i, 0))

def _part(d):
    return pl.BlockSpec((_NC, _BLK, d), lambda i: (0, i, 0))

def _full(a, b):
    return pl.BlockSpec((a, b), lambda i: (0, 0))

_DOT = dict(preferred_element_type=jnp.float32, precision=lax.Precision.HIGHEST)


def _tc_h0(x, W_in, b_in):
    def body(x_r, w_r, b_r, o_r):
        o_r[...] = jnp.dot(x_r[...], w_r[...], **_DOT) + b_r[...]
    return pl.pallas_call(
        body, grid=(_G,),
        in_specs=[_rows(128), _full(128, 128), _full(1, 128)],
        out_specs=_rows(128),
        out_shape=jax.ShapeDtypeStruct((N, 128), jnp.float32),
    )(x, W_in, b_in)


def _tc_g0(degP, h0, Wg0):
    def body(dp_r, h_r, w_r, dinv_r, g_r):
        deg = dp_r[0] + dp_r[1] + 1.0
        dinv = lax.rsqrt(deg)
        dinv_r[...] = dinv
        g_r[...] = dinv * jnp.dot(h_r[...], w_r[...], **_DOT)
    return pl.pallas_call(
        body, grid=(_G,),
        in_specs=[_part(1), _rows(128), _full(128, 128)],
        out_specs=[_rows(1), _rows(128)],
        out_shape=[jax.ShapeDtypeStruct((N, 1), jnp.float32),
                   jax.ShapeDtypeStruct((N, 128), jnp.float32)],
    )(degP, h0, Wg0)


def _tc_layer(P, g, dinv, bg, Wn, dout, want_h=True):
    def body(p_r, g_r, di_r, b_r, w_r, *outs):
        s = p_r[0] + p_r[1] - g_r[...]
        pre = di_r[...] * s + b_r[...]
        h = jnp.where(pre > 0, pre, 0.2 * pre)
        if want_h:
            outs[0][...] = h
        outs[-1][...] = di_r[...] * jnp.dot(h, w_r[...], **_DOT)
    out_specs = [_rows(128), _rows(dout)] if want_h else [_rows(dout)]
    out_shape = ([jax.ShapeDtypeStruct((N, 128), jnp.float32)] if want_h else []) + [
        jax.ShapeDtypeStruct((N, dout), jnp.float32)]
    res = pl.pallas_call(
        body, grid=(_G,),
        in_specs=[_part(128), _rows(128), _rows(1), _full(1, 128),
                  _full(128, dout)],
        out_specs=out_specs,
        out_shape=out_shape,
    )(P, g, dinv, bg, Wn)
    return res if want_h else (None, res[0])


def _tc_skips(h0, h1, h2, Ws0, bs0, Ws1, bs1, Ws2, bs2):
    def body(a_r, b_r, c_r, w0, v0, w1, v1, w2, v2, o_r):
        o_r[...] = (jnp.dot(a_r[...], w0[...], **_DOT) + v0[...]
                    + jnp.dot(b_r[...], w1[...], **_DOT) + v1[...]
                    + jnp.dot(c_r[...], w2[...], **_DOT) + v2[...])
    return pl.pallas_call(
        body, grid=(_G,),
        in_specs=[_rows(128), _rows(128), _rows(128),
                  _full(128, 64), _full(1, 64),
                  _full(128, 64), _full(1, 64),
                  _full(128, 64), _full(1, 64)],
        out_specs=_rows(64),
        out_shape=jax.ShapeDtypeStruct((N, 64), jnp.float32),
    )(h0, h1, h2, Ws0, bs0, Ws1, bs1, Ws2, bs2)


def _tc_final(PL, gl, dinv, b_lat, skips):
    def body(p_r, g_r, di_r, b_r, sk_r, o_r):
        s = p_r[0] + p_r[1] - g_r[...]
        o_r[...] = di_r[...] * s + b_r[...] + sk_r[...]
    return pl.pallas_call(
        body, grid=(_G,),
        in_specs=[_part(64), _rows(64), _rows(1), _full(1, 64), _rows(64)],
        out_specs=_rows(64),
        out_shape=jax.ShapeDtypeStruct((N, 64), jnp.float32),
    )(PL, gl, dinv, b_lat, skips)


def kernel(x, edge_index, W_in, b_in, Wg0, bg0, Wg1, bg1, Wg2, bg2,
           W_lat, b_lat, Ws0, bs0, Ws1, bs1, Ws2, bs2):
    src40 = edge_index[0].reshape(_NC * _NS, E // (_NC * _NS * 40), 40)
    dst40 = edge_index[1].reshape(_NC * _NS, E // (_NC * _NS * 40), 40)
    src80 = edge_index[0].reshape(_NC * _NS, E // (_NC * _NS * 80), 80)
    dst80 = edge_index[1].reshape(_NC * _NS, E // (_NC * _NS * 80), 80)

    degP = _sc_degree(
        edge_index[1].reshape(_NC * _NS, _CPWD, _KD)).reshape(_NC, N, 1)

    h0 = _tc_h0(x, W_in, b_in.reshape(1, 128))
    dinv, g0 = _tc_g0(degP, h0, Wg0)

    P0 = _sc_edge_agg(g0, src40, dst40, 128, 40, NB=6)
    h1, g1 = _tc_layer(P0, g0, dinv, bg0.reshape(1, 128), Wg1, 128)

    P1 = _sc_edge_agg(g1, src40, dst40, 128, 40, NB=6)
    h2, g2 = _tc_layer(P1, g1, dinv, bg1.reshape(1, 128), Wg2, 128)

    P2 = _sc_edge_agg(g2, src40, dst40, 128, 40, NB=6)
    _h3, gl = _tc_layer(P2, g2, dinv, bg2.reshape(1, 128), W_lat, 64, want_h=False)

    PL = _sc_edge_agg(gl, src80, dst80, 64, 80, NB=12, tc_tiling=False)
    skips = _tc_skips(h0, h1, h2, Ws0, bs0.reshape(1, 64),
                      Ws1, bs1.reshape(1, 64), Ws2, bs2.reshape(1, 64))
    return _tc_final(PL, gl, dinv, b_lat.reshape(1, 64), skips)
